# baseline (speedup 1.0000x reference)
import numpy as np

import jax
import jax.numpy as jnp
from jax import lax
from jax.experimental import pallas as pl
from jax.experimental.pallas import tpu as pltpu

SH = 512
BMA = 1000


def _proj_body(x_ref, w1_ref, b1_ref, w2_ref, s0_ref):
    h = lax.dot_general(x_ref[...], w1_ref[...], (((1,), (1,)), ((), ())),
                        preferred_element_type=jnp.float32)
    h = jnp.maximum(h + b1_ref[...], 0.0)
    s0_ref[...] = lax.dot_general(h, w2_ref[...], (((1,), (1,)), ((), ())),
                                  preferred_element_type=jnp.float32)


def _dot(a, b):
    return lax.dot_general(a, b, (((1,), (0,)), ((), ())),
                           precision=lax.Precision.DEFAULT,
                           preferred_element_type=jnp.float32)


def _make_sweep1_body(n, ns, hlast):
    last = ns - 1

    def dma_op(idx, rr_ref, l_ref, buf, sem, do_start):
        r = rr_ref[idx]
        roff = pl.multiple_of(r * SH, SH)
        for h, cond in ((SH, r < last), (hlast, r == last)):
            @pl.when(cond)
            def _(h=h):
                cp = pltpu.make_async_copy(
                    l_ref.at[pl.ds(roff, h), :],
                    buf.at[pl.ds(0, h), :], sem)
                if do_start:
                    cp.start()
                else:
                    cp.wait()

    def body(rr_ref, l_ref, s0_ref, s1_ref, u1_ref, buf0, buf1, sem0, sem1):
        t = pl.program_id(0)

        @pl.when(t == 0)
        def _():
            s1_ref[...] = jnp.zeros(s1_ref.shape, jnp.float32)
            dma_op(0, rr_ref, l_ref, buf0, sem0, True)

        @pl.when(jnp.logical_and(t + 1 < ns, t % 2 == 0))
        def _():
            dma_op(t + 1, rr_ref, l_ref, buf1, sem1, True)

        @pl.when(jnp.logical_and(t + 1 < ns, t % 2 == 1))
        def _():
            dma_op(t + 1, rr_ref, l_ref, buf0, sem0, True)

        def compute(buf, sem):
            r = rr_ref[t]
            dma_op(t, rr_ref, l_ref, buf, sem, False)
            roff = pl.multiple_of(r * SH, SH)
            for h, cond in ((SH, r < last), (hlast, r == last)):
                @pl.when(cond)
                def _(h=h):
                    panel = buf[...]
                    d1 = _dot(panel, s0_ref[...])
                    s1_ref[pl.ds(roff, h), :] = d1[:h, :]
                    d2 = _dot(panel, s1_ref[...])
                    u1_ref[pl.ds(roff, h), :] = d2[:h, :]

        @pl.when(t % 2 == 0)
        def _():
            compute(buf0, sem0)

        @pl.when(t % 2 == 1)
        def _():
            compute(buf1, sem1)

    return body


def _make_sweep2_body(n, ns, hlast, send, widths):
    last = ns - 1
    nw = len(widths)
    variants = [(w, SH) for w in widths] + [(send, hlast)]

    def var_cond(ws_ref, rr_ref, idx, k, h):
        wcond = ws_ref[idx] == (k if h == SH else nw)
        return wcond

    def dma_op(idx, rr_ref, ws_ref, l_ref, buf, sem, do_start):
        r = rr_ref[idx]
        roff = pl.multiple_of(r * SH, SH)
        for k, (w, h) in enumerate(variants):
            @pl.when(ws_ref[idx] == k)
            def _(w=w, h=h):
                cp = pltpu.make_async_copy(
                    l_ref.at[pl.ds(roff, h), pl.ds(send - w, w)],
                    buf.at[pl.ds(0, h), pl.ds(0, w)], sem)
                if do_start:
                    cp.start()
                else:
                    cp.wait()

    def body(rr_ref, ws_ref, l_ref, s1f_ref, s0f_ref, u1f_ref, th_ref,
             b2_ref, out_ref, buf0, buf1, sem0, sem1):
        t = pl.program_id(0)

        @pl.when(t == 0)
        def _():
            dma_op(0, rr_ref, ws_ref, l_ref, buf0, sem0, True)

        @pl.when(jnp.logical_and(t + 1 < ns, t % 2 == 0))
        def _():
            dma_op(t + 1, rr_ref, ws_ref, l_ref, buf1, sem1, True)

        @pl.when(jnp.logical_and(t + 1 < ns, t % 2 == 1))
        def _():
            dma_op(t + 1, rr_ref, ws_ref, l_ref, buf0, sem0, True)

        def epilogue(u, roff, h):
            s0 = s0f_ref[pl.ds(roff, h), :]
            s1 = s1f_ref[pl.ds(roff, h), :]
            y = (th_ref[0:1, :] * s0 + th_ref[1:2, :] * s1
                 + 2.0 * th_ref[2:3, :] * u + b2_ref[...])
            m = jnp.max(y, axis=1, keepdims=True)
            lse = jnp.log(jnp.sum(jnp.exp(y - m), axis=1, keepdims=True)) + m
            out_ref[pl.ds(roff, h), :] = y - lse

        def compute(buf, sem):
            r = rr_ref[t]
            roff = pl.multiple_of(r * SH, SH)
            mst = jnp.where(r == last, 0, (r + 1) * SH)
            for k, (w, h) in enumerate(variants):
                @pl.when(ws_ref[t] == k)
                def _(w=w, h=h):
                    dma_op(t, rr_ref, ws_ref, l_ref, buf, sem, False)
                    cstart = send - w
                    sl = s1f_ref[pl.ds(cstart, w), :]
                    rowid = lax.broadcasted_iota(
                        jnp.int32, (w, 1), 0) + cstart
                    opnd = jnp.where(rowid >= mst, sl, 0.0)
                    u = (u1f_ref[pl.ds(roff, h), :]
                         + _dot(buf[pl.ds(0, h), pl.ds(0, w)], opnd))
                    epilogue(u, roff, h)

            @pl.when(ws_ref[t] == -1)
            def _():
                epilogue(u1f_ref[pl.ds(roff, SH), :], roff, SH)

        @pl.when(t % 2 == 0)
        def _():
            compute(buf0, sem0)

        @pl.when(t % 2 == 1)
        def _():
            compute(buf1, sem1)

    return body


def kernel(x, L, W1, b1, W2, b2, thetas):
    N, F = x.shape
    H = W1.shape[0]
    C = W2.shape[0]
    ns = -(-N // SH)
    hlast = N - (ns - 1) * SH
    send = (ns - 1) * SH

    s0 = pl.pallas_call(
        _proj_body,
        grid=(N // BMA,),
        in_specs=[
            pl.BlockSpec((BMA, F), lambda i: (i, 0)),
            pl.BlockSpec((H, F), lambda i: (0, 0)),
            pl.BlockSpec((1, H), lambda i: (0, 0)),
            pl.BlockSpec((C, H), lambda i: (0, 0)),
        ],
        out_specs=pl.BlockSpec((BMA, C), lambda i: (i, 0)),
        out_shape=jax.ShapeDtypeStruct((N, C), jnp.float32),
    )(x, W1, b1.reshape(1, H), W2)

    rr1 = np.asarray([ns - 1] + list(range(ns - 1)), np.int32)

    s1, u1 = pl.pallas_call(
        _make_sweep1_body(N, ns, hlast),
        grid_spec=pltpu.PrefetchScalarGridSpec(
            num_scalar_prefetch=1,
            grid=(ns,),
            in_specs=[
                pl.BlockSpec(memory_space=pltpu.MemorySpace.HBM),
                pl.BlockSpec((N, C), lambda i, *_: (0, 0)),
            ],
            out_specs=[
                pl.BlockSpec((N, C), lambda i, *_: (0, 0)),
                pl.BlockSpec((N, C), lambda i, *_: (0, 0)),
            ],
            scratch_shapes=[
                pltpu.VMEM((SH, N), jnp.float32),
                pltpu.VMEM((SH, N), jnp.float32),
                pltpu.SemaphoreType.DMA,
                pltpu.SemaphoreType.DMA,
            ],
        ),
        out_shape=[jax.ShapeDtypeStruct((N, C), jnp.float32),
                   jax.ShapeDtypeStruct((N, C), jnp.float32)],
        compiler_params=pltpu.CompilerParams(
            dimension_semantics=("arbitrary",)),
    )(jnp.asarray(rr1), L, s0)

    widths = []
    w = 2048
    while w < send:
        widths.append(w)
        w += 2048
    widths.append(send)

    order = [ns - 1] + [r for r in range(ns - 1)]
    order = ([r for r in order if r == ns - 1 or send - (r + 1) * SH > 0]
             + [r for r in order if r != ns - 1 and send - (r + 1) * SH <= 0])
    rr2, ws2 = [], []
    for r in order:
        rr2.append(r)
        if r == ns - 1:
            ws2.append(len(widths))
        else:
            miss = send - (r + 1) * SH
            if miss <= 0:
                ws2.append(-1)
            else:
                ws2.append(min(k for k, wv in enumerate(widths)
                               if wv >= miss))
    rr2 = np.asarray(rr2, np.int32)
    ws2 = np.asarray(ws2, np.int32)

    th = jnp.broadcast_to(
        jnp.stack([thetas[0] - thetas[3], thetas[1] + thetas[2],
                   thetas[3]]).reshape(-1, 1), (3, C))

    out = pl.pallas_call(
        _make_sweep2_body(N, ns, hlast, send, widths),
        grid_spec=pltpu.PrefetchScalarGridSpec(
            num_scalar_prefetch=2,
            grid=(ns,),
            in_specs=[
                pl.BlockSpec(memory_space=pltpu.MemorySpace.HBM),
                pl.BlockSpec((N, C), lambda i, *_: (0, 0)),
                pl.BlockSpec((N, C), lambda i, *_: (0, 0)),
                pl.BlockSpec((N, C), lambda i, *_: (0, 0)),
                pl.BlockSpec((3, C), lambda i, *_: (0, 0)),
                pl.BlockSpec((1, C), lambda i, *_: (0, 0)),
            ],
            out_specs=pl.BlockSpec((N, C), lambda i, *_: (0, 0)),
            scratch_shapes=[
                pltpu.VMEM((SH, max(send, SH)), jnp.float32),
                pltpu.VMEM((SH, max(send, SH)), jnp.float32),
                pltpu.SemaphoreType.DMA,
                pltpu.SemaphoreType.DMA,
            ],
        ),
        out_shape=jax.ShapeDtypeStruct((N, C), jnp.float32),
        compiler_params=pltpu.CompilerParams(
            dimension_semantics=("arbitrary",)),
    )(jnp.asarray(rr2), jnp.asarray(ws2), L, s1, s0, u1, th,
      b2.reshape(1, C))

    return out


# speedup vs baseline: 1.2986x; 1.2986x over previous
import numpy as np

import jax
import jax.numpy as jnp
from jax import lax
from jax.experimental import pallas as pl
from jax.experimental.pallas import tpu as pltpu

BM = 400
BMA = 1000


def _proj_body(x_ref, w1_ref, b1_ref, w2_ref, s0_ref):
    h = lax.dot_general(x_ref[...], w1_ref[...], (((1,), (1,)), ((), ())),
                        preferred_element_type=jnp.float32)
    h = jnp.maximum(h + b1_ref[...], 0.0)
    s0_ref[...] = lax.dot_general(h, w2_ref[...], (((1,), (1,)), ((), ())),
                                  preferred_element_type=jnp.float32)


def _dot(a, b):
    return lax.dot_general(a, b, (((1,), (0,)), ((), ())),
                           precision=lax.Precision.DEFAULT,
                           preferred_element_type=jnp.float32)


def _sweep_body(ph_ref, rr_ref, l_ref, s0_ref, th_ref, b2_ref, out_ref,
                s1_ref):
    t = pl.program_id(0)
    r = rr_ref[t]
    roff = pl.multiple_of(r * BM, BM)

    @pl.when(ph_ref[t] == 0)
    def _():
        s1_ref[pl.ds(roff, BM), :] = _dot(l_ref[...], s0_ref[...])

    @pl.when(ph_ref[t] == 1)
    def _():
        u = _dot(l_ref[...], s1_ref[...])
        y = (th_ref[0:1, :] * s0_ref[pl.ds(roff, BM), :]
             + th_ref[1:2, :] * s1_ref[pl.ds(roff, BM), :]
             + 2.0 * th_ref[2:3, :] * u + b2_ref[...])
        m = jnp.max(y, axis=1, keepdims=True)
        lse = jnp.log(jnp.sum(jnp.exp(y - m), axis=1, keepdims=True)) + m
        out_ref[pl.ds(roff, BM), :] = y - lse


def kernel(x, L, W1, b1, W2, b2, thetas):
    N, F = x.shape
    H = W1.shape[0]
    C = W2.shape[0]
    ni = N // BM

    s0 = pl.pallas_call(
        _proj_body,
        grid=(N // BMA,),
        in_specs=[
            pl.BlockSpec((BMA, F), lambda i: (i, 0)),
            pl.BlockSpec((H, F), lambda i: (0, 0)),
            pl.BlockSpec((1, H), lambda i: (0, 0)),
            pl.BlockSpec((C, H), lambda i: (0, 0)),
        ],
        out_specs=pl.BlockSpec((BMA, C), lambda i: (i, 0)),
        out_shape=jax.ShapeDtypeStruct((N, C), jnp.float32),
    )(x, W1, b1.reshape(1, H), W2)

    ph = np.asarray([0] * ni + [1] * ni, np.int32)
    rr = np.asarray(list(range(ni)) * 2, np.int32)

    th = jnp.broadcast_to(
        jnp.stack([thetas[0] - thetas[3], thetas[1] + thetas[2],
                   thetas[3]]).reshape(-1, 1), (3, C))

    out = pl.pallas_call(
        _sweep_body,
        grid_spec=pltpu.PrefetchScalarGridSpec(
            num_scalar_prefetch=2,
            grid=(2 * ni,),
            in_specs=[
                pl.BlockSpec((BM, N), lambda i, ph, rr: (rr[i], 0)),
                pl.BlockSpec((N, C), lambda i, *_: (0, 0)),
                pl.BlockSpec((3, C), lambda i, *_: (0, 0)),
                pl.BlockSpec((1, C), lambda i, *_: (0, 0)),
            ],
            out_specs=pl.BlockSpec((N, C), lambda i, *_: (0, 0)),
            scratch_shapes=[pltpu.VMEM((N, C), jnp.float32)],
        ),
        out_shape=jax.ShapeDtypeStruct((N, C), jnp.float32),
        compiler_params=pltpu.CompilerParams(
            dimension_semantics=("arbitrary",)),
    )(jnp.asarray(ph), jnp.asarray(rr), L, s0, th, b2.reshape(1, C))

    return out


# speedup vs baseline: 1.3120x; 1.0104x over previous
import numpy as np

import jax
import jax.numpy as jnp
from jax import lax
from jax.experimental import pallas as pl
from jax.experimental.pallas import tpu as pltpu

BM = 400
BMA = 1000


def _dot(a, b):
    return lax.dot_general(a, b, (((1,), (0,)), ((), ())),
                           precision=lax.Precision.DEFAULT,
                           preferred_element_type=jnp.float32)


def _sweep_body(ph_ref, rr_ref, l_ref, x_ref, w1_ref, b1_ref, w2_ref,
                th_ref, b2_ref, out_ref, s0_ref, s1_ref):
    t = pl.program_id(0)
    r = rr_ref[t]

    @pl.when(ph_ref[t] == 2)
    def _():
        poff = pl.multiple_of(r * BMA, BMA)
        h = lax.dot_general(x_ref[...], w1_ref[...],
                            (((1,), (1,)), ((), ())),
                            preferred_element_type=jnp.float32)
        h = jnp.maximum(h + b1_ref[...], 0.0)
        s0_ref[pl.ds(poff, BMA), :] = lax.dot_general(
            h, w2_ref[...], (((1,), (1,)), ((), ())),
            preferred_element_type=jnp.float32)

    @pl.when(ph_ref[t] == 0)
    def _():
        roff = pl.multiple_of(r * BM, BM)
        s1_ref[pl.ds(roff, BM), :] = _dot(l_ref[...], s0_ref[...])

    @pl.when(ph_ref[t] == 1)
    def _():
        roff = pl.multiple_of(r * BM, BM)
        u = _dot(l_ref[...], s1_ref[...])
        y = (th_ref[0:1, :] * s0_ref[pl.ds(roff, BM), :]
             + th_ref[1:2, :] * s1_ref[pl.ds(roff, BM), :]
             + 2.0 * th_ref[2:3, :] * u + b2_ref[...])
        m = jnp.max(y, axis=1, keepdims=True)
        lse = jnp.log(jnp.sum(jnp.exp(y - m), axis=1, keepdims=True)) + m
        out_ref[pl.ds(roff, BM), :] = y - lse


def kernel(x, L, W1, b1, W2, b2, thetas):
    N, F = x.shape
    H = W1.shape[0]
    C = W2.shape[0]
    ni = N // BM
    np_ = N // BMA

    ph = np.asarray([2] * np_ + [0] * ni + [1] * ni, np.int32)
    rr = np.asarray(list(range(np_)) + list(range(ni)) * 2, np.int32)

    th = jnp.broadcast_to(
        jnp.stack([thetas[0] - thetas[3], thetas[1] + thetas[2],
                   thetas[3]]).reshape(-1, 1), (3, C))

    out = pl.pallas_call(
        _sweep_body,
        grid_spec=pltpu.PrefetchScalarGridSpec(
            num_scalar_prefetch=2,
            grid=(np_ + 2 * ni,),
            in_specs=[
                pl.BlockSpec((BM, N),
                             lambda i, ph, rr: (jnp.where(ph[i] == 2, 0,
                                                          rr[i]), 0)),
                pl.BlockSpec((BMA, F),
                             lambda i, ph, rr: (jnp.where(ph[i] == 2,
                                                          rr[i], 0), 0)),
                pl.BlockSpec((H, F), lambda i, *_: (0, 0)),
                pl.BlockSpec((1, H), lambda i, *_: (0, 0)),
                pl.BlockSpec((C, H), lambda i, *_: (0, 0)),
                pl.BlockSpec((3, C), lambda i, *_: (0, 0)),
                pl.BlockSpec((1, C), lambda i, *_: (0, 0)),
            ],
            out_specs=pl.BlockSpec((N, C), lambda i, *_: (0, 0)),
            scratch_shapes=[pltpu.VMEM((N, C), jnp.float32),
                            pltpu.VMEM((N, C), jnp.float32)],
        ),
        out_shape=jax.ShapeDtypeStruct((N, C), jnp.float32),
        compiler_params=pltpu.CompilerParams(
            dimension_semantics=("arbitrary",)),
    )(jnp.asarray(ph), jnp.asarray(rr), L, x, W1, b1.reshape(1, H), W2, th,
      b2.reshape(1, C))

    return out
